# Initial kernel scaffold; baseline (speedup 1.0000x reference)
import jax
import jax.numpy as jnp
from jax.experimental import pallas as pl


def kernel(density, rgb, ray_id):
    raise NotImplementedError("write your pallas kernel here")


# speedup vs baseline: 9.7978x; 9.7978x over previous
import functools
import numpy as np
import jax
import jax.numpy as jnp
from jax import lax
from jax.experimental import pallas as pl
from jax.experimental.pallas import tpu as pltpu
from jax.experimental.pallas import tpu_sc as plsc

N_RAYS = 8192
NP_TOT = 2097152
INTERVAL = 0.5
SHIFT = float(np.log(1.0 / (1.0 - 0.01) - 1.0))
LN2 = float(np.log(2.0))
SQRT2 = float(np.sqrt(2.0))
PC = [3.342326871519363e-08, 1.0000030986470902, -0.5000129330593485,
      0.33304812395021915, -0.24911210645484655, 0.206117852396594,
      -0.18627697325290674, 0.11448435452372649]

NW = 32
CH = NP_TOT // NW
NSTAGE = 32
STAGE = CH // NSTAGE
NVEC = STAGE // 16
RPW = N_RAYS // NW

_mesh = plsc.VectorSubcoreMesh(core_axis_name="c", subcore_axis_name="s")
_cparams = pltpu.CompilerParams(needs_layout_passes=False,
                                use_tc_tiling_on_sc=False)


def _logt_of_density(den):
  x = den + jnp.float32(SHIFT)
  e = jnp.exp(x)
  u = jnp.float32(1.0) + e
  bits = plsc.bitcast(u, jnp.int32)
  ei = (bits >> 23) - 127
  m = plsc.bitcast((bits & 0x007FFFFF) | 0x3F800000, jnp.float32)
  adj = m > jnp.float32(SQRT2)
  m2 = jnp.where(adj, m * jnp.float32(0.5), m)
  ef = ei.astype(jnp.float32) + jnp.where(adj, jnp.float32(1.0),
                                          jnp.float32(0.0))
  z = m2 - jnp.float32(1.0)
  p = jnp.float32(PC[7])
  for k in range(6, -1, -1):
    p = p * z + jnp.float32(PC[k])
  lnu = ef * jnp.float32(LN2) + p
  return jnp.float32(-INTERVAL) * lnu


def _sigmoid(v):
  return jnp.float32(1.0) / (jnp.float32(1.0) + jnp.exp(-v))


@functools.partial(
    pl.kernel,
    out_type=jax.ShapeDtypeStruct((NW, 4, N_RAYS), jnp.float32),
    mesh=_mesh,
    scratch_types=[
        pltpu.VMEM((2 * STAGE,), jnp.float32),
        pltpu.VMEM((2 * STAGE,), jnp.int32),
        pltpu.VMEM((2 * STAGE * 3,), jnp.float32),
        pltpu.VMEM((N_RAYS,), jnp.float32),
        pltpu.VMEM((N_RAYS,), jnp.float32),
        pltpu.VMEM((N_RAYS,), jnp.float32),
        pltpu.VMEM((N_RAYS,), jnp.float32),
        pltpu.VMEM((64,), jnp.float32),
        pltpu.SemaphoreType.DMA,
    ],
    compiler_params=_cparams,
)
def _phase1(den_hbm, rgb_hbm, id_hbm, acc_hbm,
            den_v, id_v, rgb_v, s0_v, s1_v, s2_v, r_v, tmp_v, sem):
  wid = lax.axis_index("c") * 16 + lax.axis_index("s")
  base_smp = wid * CH

  iota = lax.iota(jnp.int32, 16)
  zero16 = jnp.zeros((16,), jnp.float32)
  one16 = jnp.ones((16,), jnp.float32)
  m1_16 = jnp.full((16,), -1, jnp.int32)

  def _init(i, _):
    sl = pl.ds(i * 16, 16)
    s0_v[sl] = zero16
    s1_v[sl] = zero16
    s2_v[sl] = zero16
    r_v[sl] = one16
    return 0
  lax.fori_loop(0, N_RAYS // 16, _init, 0)

  def _starts(s, buf):
    off = base_smp + s * STAGE
    pltpu.make_async_copy(den_hbm.at[pl.ds(off, STAGE)],
                          den_v.at[pl.ds(buf * STAGE, STAGE)], sem).start()
    pltpu.make_async_copy(id_hbm.at[pl.ds(off, STAGE)],
                          id_v.at[pl.ds(buf * STAGE, STAGE)], sem).start()
    pltpu.make_async_copy(rgb_hbm.at[pl.ds(off * 3, STAGE * 3)],
                          rgb_v.at[pl.ds(buf * STAGE * 3, STAGE * 3)],
                          sem).start()

  def _waits(s, buf):
    off = base_smp + s * STAGE
    pltpu.make_async_copy(den_hbm.at[pl.ds(off, STAGE)],
                          den_v.at[pl.ds(buf * STAGE, STAGE)], sem).wait()
    pltpu.make_async_copy(id_hbm.at[pl.ds(off, STAGE)],
                          id_v.at[pl.ds(buf * STAGE, STAGE)], sem).wait()
    pltpu.make_async_copy(rgb_hbm.at[pl.ds(off * 3, STAGE * 3)],
                          rgb_v.at[pl.ds(buf * STAGE * 3, STAGE * 3)],
                          sem).wait()

  _starts(0, 0)

  def _stage(s, carry):
    buf = s % 2
    _waits(s, buf)

    @pl.when(s + 1 < NSTAGE)
    def _():
      _starts(s + 1, (s + 1) % 2)

    dbase = buf * STAGE
    rbase = buf * STAGE * 3

    def _vec(v, carry):
      carry_c, carry_b, prev_id = carry
      off = dbase + v * 16
      den = den_v[pl.ds(off, 16)]
      idv = id_v[pl.ds(off, 16)]
      roff = rbase + v * 48
      rg0 = plsc.load_gather(rgb_v, [roff + 3 * iota])
      rg1 = plsc.load_gather(rgb_v, [roff + 3 * iota + 1])
      rg2 = plsc.load_gather(rgb_v, [roff + 3 * iota + 2])

      lt = _logt_of_density(den)
      c = plsc.cumsum(lt)
      cglob = c + carry_c
      ex = cglob - lt

      pidg = plsc.load_gather(id_v, [off + jnp.maximum(iota - 1, 0)])
      pid = jnp.where(iota == 0, prev_id, pidg)
      idng = plsc.load_gather(id_v, [off + jnp.minimum(iota + 1, 15)])
      change = idv != pid
      st2 = plsc.cummax(jnp.where(change, iota, m1_16))
      startv = jnp.maximum(st2, 0)
      gidx = jnp.maximum(startv - 1, 0)
      has_bef = startv >= 1
      tmp_v[pl.ds(0, 16)] = cglob
      gbase = plsc.load_gather(tmp_v, [gidx])
      basev = jnp.where(st2 >= 0, jnp.where(has_bef, gbase, carry_c), carry_b)

      texc = jnp.exp(ex - basev)
      tinc = jnp.exp(cglob - basev)
      w = texc - tinc

      wc0 = w * _sigmoid(rg0)
      wc1 = w * _sigmoid(rg1)
      wc2 = w * _sigmoid(rg2)
      cs0 = plsc.cumsum(wc0)
      cs1 = plsc.cumsum(wc1)
      cs2 = plsc.cumsum(wc2)
      tmp_v[pl.ds(16, 16)] = cs0
      tmp_v[pl.ds(32, 16)] = cs1
      tmp_v[pl.ds(48, 16)] = cs2
      ss0 = cs0 - jnp.where(has_bef, plsc.load_gather(tmp_v, [16 + gidx]),
                            zero16)
      ss1 = cs1 - jnp.where(has_bef, plsc.load_gather(tmp_v, [32 + gidx]),
                            zero16)
      ss2 = cs2 - jnp.where(has_bef, plsc.load_gather(tmp_v, [48 + gidx]),
                            zero16)

      maskst = (idng != idv) | (iota == 15)

      plsc.addupdate_scatter(s0_v, [idv], ss0, mask=maskst)
      plsc.addupdate_scatter(s1_v, [idv], ss1, mask=maskst)
      plsc.addupdate_scatter(s2_v, [idv], ss2, mask=maskst)
      plsc.store_scatter(r_v, [idv], tinc, mask=maskst)

      return (jnp.min(cglob), jnp.min(basev), jnp.max(idv))

    return lax.fori_loop(0, NVEC, _vec, carry)

  carry0 = (jnp.float32(0.0), jnp.float32(0.0), jnp.int32(-1))
  lax.fori_loop(0, NSTAGE, _stage, carry0)

  pltpu.sync_copy(s0_v, acc_hbm.at[wid, 0])
  pltpu.sync_copy(s1_v, acc_hbm.at[wid, 1])
  pltpu.sync_copy(s2_v, acc_hbm.at[wid, 2])
  pltpu.sync_copy(r_v, acc_hbm.at[wid, 3])


@functools.partial(
    pl.kernel,
    out_type=jax.ShapeDtypeStruct((3, N_RAYS), jnp.float32),
    mesh=_mesh,
    scratch_types=[
        pltpu.VMEM((NW, 4, RPW), jnp.float32),
        pltpu.VMEM((3, RPW), jnp.float32),
    ],
    compiler_params=_cparams,
)
def _phase2(acc_hbm, out_hbm, acc_v, out_v):
  wid = lax.axis_index("c") * 16 + lax.axis_index("s")
  rbase = wid * RPW
  pltpu.sync_copy(acc_hbm.at[:, :, pl.ds(rbase, RPW)], acc_v)

  def _blk(j, _):
    sl = pl.ds(j * 16, 16)

    def _chunk(k, carry):
      o0, o1, o2, t = carry
      o0 = o0 + t * acc_v[k, 0, sl]
      o1 = o1 + t * acc_v[k, 1, sl]
      o2 = o2 + t * acc_v[k, 2, sl]
      t = t * acc_v[k, 3, sl]
      return (o0, o1, o2, t)

    z = jnp.zeros((16,), jnp.float32)
    o0, o1, o2, t = lax.fori_loop(0, NW, _chunk, (z, z, z,
                                                  jnp.ones((16,), jnp.float32)))
    out_v[0, sl] = o0 + t
    out_v[1, sl] = o1 + t
    out_v[2, sl] = o2 + t
    return 0

  lax.fori_loop(0, RPW // 16, _blk, 0)
  pltpu.sync_copy(out_v, out_hbm.at[:, pl.ds(rbase, RPW)])


def kernel(density, rgb, ray_id):
  acc = _phase1(density, rgb.reshape(-1), ray_id)
  out = _phase2(acc)
  return out.T


# speedup vs baseline: 78.9442x; 8.0574x over previous
import functools
import numpy as np
import jax
import jax.numpy as jnp
from jax import lax
from jax.experimental import pallas as pl
from jax.experimental.pallas import tpu as pltpu
from jax.experimental.pallas import tpu_sc as plsc

N_RAYS = 8192
NP_TOT = 2097152
INTERVAL = 0.5
SHIFT = float(np.log(1.0 / (1.0 - 0.01) - 1.0))
LN2 = float(np.log(2.0))
SQRT2 = float(np.sqrt(2.0))
PC = [3.342326871519363e-08, 1.0000030986470902, -0.5000129330593485,
      0.33304812395021915, -0.24911210645484655, 0.206117852396594,
      -0.18627697325290674, 0.11448435452372649]

NW = 32
CH = NP_TOT // NW
NSTAGE = 32
STAGE = CH // NSTAGE
NVEC = STAGE // 16
RPW = N_RAYS // NW

_mesh = plsc.VectorSubcoreMesh(core_axis_name="c", subcore_axis_name="s")
_cparams = pltpu.CompilerParams(needs_layout_passes=False,
                                use_tc_tiling_on_sc=False)


def _logt_of_density(den):
  x = den + jnp.float32(SHIFT)
  e = jnp.exp(x)
  u = jnp.float32(1.0) + e
  bits = plsc.bitcast(u, jnp.int32)
  ei = (bits >> 23) - 127
  m = plsc.bitcast((bits & 0x007FFFFF) | 0x3F800000, jnp.float32)
  adj = m > jnp.float32(SQRT2)
  m2 = jnp.where(adj, m * jnp.float32(0.5), m)
  ef = ei.astype(jnp.float32) + jnp.where(adj, jnp.float32(1.0),
                                          jnp.float32(0.0))
  z = m2 - jnp.float32(1.0)
  p = jnp.float32(PC[7])
  for k in range(6, -1, -1):
    p = p * z + jnp.float32(PC[k])
  lnu = ef * jnp.float32(LN2) + p
  return jnp.float32(-INTERVAL) * lnu


def _sigmoid(v):
  return jnp.float32(1.0) / (jnp.float32(1.0) + jnp.exp(-v))


@functools.partial(
    pl.kernel,
    out_type=jax.ShapeDtypeStruct((NW, 4, N_RAYS), jnp.float32),
    mesh=_mesh,
    scratch_types=[
        pltpu.VMEM((2 * STAGE,), jnp.float32),
        pltpu.VMEM((2 * STAGE,), jnp.int32),
        pltpu.VMEM((2 * STAGE,), jnp.float32),
        pltpu.VMEM((2 * STAGE,), jnp.float32),
        pltpu.VMEM((2 * STAGE,), jnp.float32),
        pltpu.VMEM((N_RAYS,), jnp.float32),
        pltpu.VMEM((N_RAYS,), jnp.float32),
        pltpu.VMEM((N_RAYS,), jnp.float32),
        pltpu.VMEM((N_RAYS,), jnp.float32),
        pltpu.VMEM((64,), jnp.float32),
        pltpu.SemaphoreType.DMA,
    ],
    compiler_params=_cparams,
)
def _phase1(den_hbm, cr_hbm, cg_hbm, cb_hbm, id_hbm, acc_hbm,
            den_v, id_v, cr_v, cg_v, cb_v, s0_v, s1_v, s2_v, r_v, tmp_v, sem):
  wid = lax.axis_index("c") * 16 + lax.axis_index("s")
  base_smp = wid * CH

  iota = lax.iota(jnp.int32, 16)
  zero16 = jnp.zeros((16,), jnp.float32)
  one16 = jnp.ones((16,), jnp.float32)
  m1_16 = jnp.full((16,), -1, jnp.int32)

  def _init(i, _):
    sl = pl.ds(i * 16, 16)
    s0_v[sl] = zero16
    s1_v[sl] = zero16
    s2_v[sl] = zero16
    r_v[sl] = one16
    return 0
  lax.fori_loop(0, N_RAYS // 16, _init, 0)

  def _copies(s, buf):
    off = base_smp + s * STAGE
    dst = pl.ds(buf * STAGE, STAGE)
    src = pl.ds(off, STAGE)
    return [
        pltpu.make_async_copy(den_hbm.at[src], den_v.at[dst], sem),
        pltpu.make_async_copy(id_hbm.at[src], id_v.at[dst], sem),
        pltpu.make_async_copy(cr_hbm.at[src], cr_v.at[dst], sem),
        pltpu.make_async_copy(cg_hbm.at[src], cg_v.at[dst], sem),
        pltpu.make_async_copy(cb_hbm.at[src], cb_v.at[dst], sem),
    ]

  def _starts(s, buf):
    for c in _copies(s, buf):
      c.start()

  def _waits(s, buf):
    for c in _copies(s, buf):
      c.wait()

  _starts(0, 0)

  def _stage(s, carry):
    buf = s % 2
    _waits(s, buf)

    @pl.when(s + 1 < NSTAGE)
    def _():
      _starts(s + 1, (s + 1) % 2)

    dbase = buf * STAGE

    def _vec(v, carry):
      carry_c, carry_b, prev_id = carry
      off = dbase + v * 16
      den = den_v[pl.ds(off, 16)]
      idv = id_v[pl.ds(off, 16)]
      rg0 = cr_v[pl.ds(off, 16)]
      rg1 = cg_v[pl.ds(off, 16)]
      rg2 = cb_v[pl.ds(off, 16)]

      lt = _logt_of_density(den)
      c = plsc.cumsum(lt)
      cglob = c + carry_c
      ex = cglob - lt

      pidg = plsc.load_gather(id_v, [off + jnp.maximum(iota - 1, 0)])
      pid = jnp.where(iota == 0, prev_id, pidg)
      idng = plsc.load_gather(id_v, [off + jnp.minimum(iota + 1, 15)])
      change = idv != pid
      st2 = plsc.cummax(jnp.where(change, iota, m1_16))
      startv = jnp.maximum(st2, 0)
      gidx = jnp.maximum(startv - 1, 0)
      has_bef = startv >= 1
      tmp_v[pl.ds(0, 16)] = cglob
      gbase = plsc.load_gather(tmp_v, [gidx])
      basev = jnp.where(st2 >= 0, jnp.where(has_bef, gbase, carry_c), carry_b)

      texc = jnp.exp(ex - basev)
      tinc = jnp.exp(cglob - basev)
      w = texc - tinc

      wc0 = w * _sigmoid(rg0)
      wc1 = w * _sigmoid(rg1)
      wc2 = w * _sigmoid(rg2)
      cs0 = plsc.cumsum(wc0)
      cs1 = plsc.cumsum(wc1)
      cs2 = plsc.cumsum(wc2)
      tmp_v[pl.ds(16, 16)] = cs0
      tmp_v[pl.ds(32, 16)] = cs1
      tmp_v[pl.ds(48, 16)] = cs2
      ss0 = cs0 - jnp.where(has_bef, plsc.load_gather(tmp_v, [16 + gidx]),
                            zero16)
      ss1 = cs1 - jnp.where(has_bef, plsc.load_gather(tmp_v, [32 + gidx]),
                            zero16)
      ss2 = cs2 - jnp.where(has_bef, plsc.load_gather(tmp_v, [48 + gidx]),
                            zero16)

      maskst = (idng != idv) | (iota == 15)

      plsc.addupdate_scatter(s0_v, [idv], ss0, mask=maskst)
      plsc.addupdate_scatter(s1_v, [idv], ss1, mask=maskst)
      plsc.addupdate_scatter(s2_v, [idv], ss2, mask=maskst)
      plsc.store_scatter(r_v, [idv], tinc, mask=maskst)

      return (jnp.min(cglob), jnp.min(basev), jnp.max(idv))

    return lax.fori_loop(0, NVEC, _vec, carry)

  carry0 = (jnp.float32(0.0), jnp.float32(0.0), jnp.int32(-1))
  lax.fori_loop(0, NSTAGE, _stage, carry0)

  pltpu.sync_copy(s0_v, acc_hbm.at[wid, 0])
  pltpu.sync_copy(s1_v, acc_hbm.at[wid, 1])
  pltpu.sync_copy(s2_v, acc_hbm.at[wid, 2])
  pltpu.sync_copy(r_v, acc_hbm.at[wid, 3])


@functools.partial(
    pl.kernel,
    out_type=jax.ShapeDtypeStruct((3, N_RAYS), jnp.float32),
    mesh=_mesh,
    scratch_types=[
        pltpu.VMEM((NW, 4, RPW), jnp.float32),
        pltpu.VMEM((3, RPW), jnp.float32),
    ],
    compiler_params=_cparams,
)
def _phase2(acc_hbm, out_hbm, acc_v, out_v):
  wid = lax.axis_index("c") * 16 + lax.axis_index("s")
  rbase = wid * RPW
  pltpu.sync_copy(acc_hbm.at[:, :, pl.ds(rbase, RPW)], acc_v)

  def _blk(j, _):
    sl = pl.ds(j * 16, 16)

    def _chunk(k, carry):
      o0, o1, o2, t = carry
      o0 = o0 + t * acc_v[k, 0, sl]
      o1 = o1 + t * acc_v[k, 1, sl]
      o2 = o2 + t * acc_v[k, 2, sl]
      t = t * acc_v[k, 3, sl]
      return (o0, o1, o2, t)

    z = jnp.zeros((16,), jnp.float32)
    o0, o1, o2, t = lax.fori_loop(0, NW, _chunk, (z, z, z,
                                                  jnp.ones((16,), jnp.float32)))
    out_v[0, sl] = o0 + t
    out_v[1, sl] = o1 + t
    out_v[2, sl] = o2 + t
    return 0

  lax.fori_loop(0, RPW // 16, _blk, 0)
  pltpu.sync_copy(out_v, out_hbm.at[:, pl.ds(rbase, RPW)])


def kernel(density, rgb, ray_id):
  acc = _phase1(density, rgb[:, 0], rgb[:, 1], rgb[:, 2], ray_id)
  out = _phase2(acc)
  return out.T
